# baseline (speedup 1.0000x reference)
import functools

import jax
import jax.numpy as jnp
from jax import lax
from jax.experimental import pallas as pl
from jax.experimental.pallas import tpu as pltpu
from jax.experimental.pallas import tpu_sc as plsc

_N = 10000
_E = 160000
_NPAD = 10240
_NW = 32
_CH = 128
_NCH = 40
_EPW = _CH * _NCH
_EPAD = _EPW * _NW
_RPW = _NPAD // 16


def _mesh():
    return plsc.VectorSubcoreMesh(core_axis_name="c", subcore_axis_name="s")


def _make_agg(feat):
    @functools.partial(
        pl.kernel,
        mesh=_mesh(),
        compiler_params=pltpu.CompilerParams(use_tc_tiling_on_sc=False),
        out_type=jax.ShapeDtypeStruct((2, _NPAD, feat), jnp.float32),
        scratch_types=[pltpu.VMEM((2, _CH), jnp.int32)] * 8 + [
            pltpu.VMEM((_CH, feat), jnp.float32),
            pltpu.VMEM((_CH, feat), jnp.float32),
            pltpu.VMEM((_CH, feat), jnp.float32),
            pltpu.VMEM((_CH, feat), jnp.float32),
            pltpu.VMEM_SHARED((_NPAD, feat), jnp.float32),
            pltpu.VMEM_SHARED((_NPAD, feat), jnp.float32),
        ] + [pltpu.SemaphoreType.DMA] * 16,
    )
    def agg(e_hbm, tab_hbm, out_hbm,
            i0, i1, i2, i3, i4, i5, i6, i7, r0, r1, r2, r3, tab_sh, acc,
            si0, si1, si2, si3, si4, si5, si6, si7,
            sg0, sg1, sg2, sg3, ss0, ss1, ss2, ss3):
        c = lax.axis_index("c")
        s = lax.axis_index("s")
        wid = s * 2 + c
        idxb = [i0, i1, i2, i3, i4, i5, i6, i7]
        rowsb = [r0, r1, r2, r3]
        si = [si0, si1, si2, si3, si4, si5, si6, si7]
        sg = [sg0, sg1, sg2, sg3]
        ss = [ss0, ss1, ss2, ss3]

        def start_idx(k, b):
            pltpu.async_copy(e_hbm.at[wid, k], idxb[b], si[b])

        def wait_idx(k, b):
            pltpu.make_async_copy(e_hbm.at[wid, k], idxb[b], si[b]).wait()

        def start_gather(b, rb):
            pltpu.async_copy(tab_sh.at[idxb[b].at[0]], rowsb[rb], sg[rb])

        def wait_gather(b, rb):
            pltpu.make_async_copy(
                tab_sh.at[idxb[b].at[0]], rowsb[rb], sg[rb]).wait()

        def start_scat(b, rb):
            pltpu.async_copy(rowsb[rb], acc.at[idxb[b].at[1]], ss[rb],
                             add=True)

        def wait_scat(b, rb):
            pltpu.make_async_copy(
                rowsb[rb], acc.at[idxb[b].at[1]], ss[rb]).wait()

        def step(k, b, rb, has_next, has_prev, has_pref):
            wait_gather(b, rb)
            start_scat(b, rb)
            if has_next:
                if has_prev:
                    wait_scat((b + 7) % 8, (rb + 3) % 4)
                wait_idx(k + 3, (b + 3) % 8)
                start_gather((b + 3) % 8, (rb + 3) % 4)
                if has_pref:
                    start_idx(k + 7, (b + 7) % 8)

        pltpu.sync_copy(tab_hbm.at[pl.ds(s * _RPW, _RPW)],
                        tab_sh.at[pl.ds(s * _RPW, _RPW)])

        def _zero(i, carry):
            for j in range(feat // 16):
                r0[i, pl.ds(j * 16, 16)] = jnp.zeros((16,), jnp.float32)
            return carry

        lax.fori_loop(0, _CH, _zero, 0)
        for t in range(_RPW // _CH):
            pltpu.sync_copy(r0, acc.at[pl.ds(s * _RPW + t * _CH, _CH)])
        for b in range(7):
            start_idx(b, b)
        plsc.subcore_barrier()
        for k in range(3):
            wait_idx(k, k)
            start_gather(k, k)

        step(0, 0, 0, True, False, True)
        for k in range(1, 8):
            step(k, k % 8, k % 4, True, True, k + 7 < _NCH)

        def _mid(t, carry):
            k0 = 8 * t
            for i in range(8):
                step(k0 + i, i, i % 4, True, True, True)
            return carry

        lax.fori_loop(1, _NCH // 8 - 1, _mid, 0)

        for k in range(_NCH - 8, _NCH):
            step(k, k % 8, k % 4, k + 3 < _NCH, k + 3 < _NCH, k + 7 < _NCH)
        for k in range(_NCH - 4, _NCH):
            wait_scat(k % 8, k % 4)

        plsc.subcore_barrier()
        pltpu.sync_copy(acc.at[pl.ds(s * _RPW, _RPW)],
                        out_hbm.at[c, pl.ds(s * _RPW, _RPW)])

    return agg


_agg64 = _make_agg(64)
_agg32 = _make_agg(32)


@functools.partial(
    pl.kernel,
    mesh=_mesh(),
    compiler_params=pltpu.CompilerParams(use_tc_tiling_on_sc=False),
    out_type=jax.ShapeDtypeStruct((2, _NPAD, 16), jnp.float32),
    scratch_types=[
        pltpu.VMEM((2, _CH), jnp.int32),
        pltpu.VMEM((2, _CH), jnp.int32),
        pltpu.VMEM((_CH, 16), jnp.float32),
        pltpu.VMEM_SHARED((_NPAD, 16), jnp.float32),
    ] + [pltpu.SemaphoreType.DMA] * 2,
)
def _deg(e_hbm, out_hbm, i0, i1, rows, acc, si0, si1):
    c = lax.axis_index("c")
    s = lax.axis_index("s")
    wid = s * 2 + c
    idxb = [i0, i1]
    si = [si0, si1]

    def _zero(i, carry):
        rows[i, pl.ds(0, 16)] = jnp.zeros((16,), jnp.float32)
        return carry

    lax.fori_loop(0, _CH, _zero, 0)
    for t in range(_RPW // _CH):
        pltpu.sync_copy(rows, acc.at[pl.ds(s * _RPW + t * _CH, _CH)])

    e0 = jnp.where(lax.iota(jnp.int32, 16) == 0, 1.0, 0.0).astype(jnp.float32)

    def _fill(i, carry):
        rows[i, pl.ds(0, 16)] = e0
        return carry

    lax.fori_loop(0, _CH, _fill, 0)
    pltpu.async_copy(e_hbm.at[wid, 0], i0, si0)
    pltpu.async_copy(e_hbm.at[wid, 1], i1, si1)
    plsc.subcore_barrier()

    def _chunk(j, carry):
        for b in range(2):
            k = 2 * j + b
            pltpu.make_async_copy(e_hbm.at[wid, k], idxb[b], si[b]).wait()
            pltpu.sync_copy(rows, acc.at[idxb[b].at[1]], add=True)
            nxt = k + 2

            @pl.when(nxt < _NCH)
            def _():
                pltpu.async_copy(e_hbm.at[wid, nxt], idxb[b], si[b])

        return carry

    lax.fori_loop(0, _NCH // 2, _chunk, 0)
    plsc.subcore_barrier()
    pltpu.sync_copy(acc.at[pl.ds(s * _RPW, _RPW)],
                    out_hbm.at[c, pl.ds(s * _RPW, _RPW)])


_RB = 1024


def _tc_xw1(x_pad, w1):
    def body(x_ref, w_ref, o_ref):
        o_ref[...] = jnp.dot(x_ref[...], w_ref[...],
                             preferred_element_type=jnp.float32)

    return pl.pallas_call(
        body,
        grid=(_NPAD // _RB,),
        in_specs=[
            pl.BlockSpec((_RB, 128), lambda i: (i, 0)),
            pl.BlockSpec((128, 64), lambda i: (0, 0)),
        ],
        out_specs=pl.BlockSpec((_RB, 64), lambda i: (i, 0)),
        out_shape=jax.ShapeDtypeStruct((_NPAD, 64), jnp.float32),
    )(x_pad, w1)


def _tc_scale1(deg2, xw1):
    def body(deg_ref, xw_ref, y_ref, dinv_ref):
        deg = deg_ref[0, :, 0:1] + deg_ref[1, :, 0:1] + 1.0
        dinv = lax.rsqrt(deg)
        y_ref[...] = dinv * xw_ref[...]
        dinv_ref[...] = dinv

    return pl.pallas_call(
        body,
        grid=(_NPAD // _RB,),
        in_specs=[
            pl.BlockSpec((2, _RB, 16), lambda i: (0, i, 0)),
            pl.BlockSpec((_RB, 64), lambda i: (i, 0)),
        ],
        out_specs=[
            pl.BlockSpec((_RB, 64), lambda i: (i, 0)),
            pl.BlockSpec((_RB, 1), lambda i: (i, 0)),
        ],
        out_shape=[
            jax.ShapeDtypeStruct((_NPAD, 64), jnp.float32),
            jax.ShapeDtypeStruct((_NPAD, 1), jnp.float32),
        ],
    )(deg2, xw1)


def _tc_elt(acc, y1, dinv, b1):
    def body(acc_ref, y_ref, dinv_ref, b_ref, o_ref):
        t = dinv_ref[...] * (acc_ref[0] + acc_ref[1] + y_ref[...]) + b_ref[...]
        o_ref[...] = dinv_ref[...] * jnp.maximum(t, 0.0)

    return pl.pallas_call(
        body,
        grid=(_NPAD // _RB,),
        in_specs=[
            pl.BlockSpec((2, _RB, 64), lambda i: (0, i, 0)),
            pl.BlockSpec((_RB, 64), lambda i: (i, 0)),
            pl.BlockSpec((_RB, 1), lambda i: (i, 0)),
            pl.BlockSpec((1, 64), lambda i: (0, 0)),
        ],
        out_specs=pl.BlockSpec((_RB, 64), lambda i: (i, 0)),
        out_shape=jax.ShapeDtypeStruct((_NPAD, 64), jnp.float32),
    )(acc, y1, dinv, b1)


def _tc_latent(acc, u2, dinv, bcat, wcat):
    def body(acc_ref, u_ref, dinv_ref, b_ref, w_ref, mu_ref, lv_ref, u3_ref):
        t = jnp.dot(acc_ref[0] + acc_ref[1] + u_ref[...], w_ref[...],
                    preferred_element_type=jnp.float32)
        outcat = dinv_ref[...] * t + b_ref[...]
        mu = outcat[:, 0:32]
        mu_ref[...] = mu
        lv_ref[...] = outcat[:, 32:64]
        u3_ref[...] = dinv_ref[...] * mu

    return pl.pallas_call(
        body,
        grid=(_NPAD // _RB,),
        in_specs=[
            pl.BlockSpec((2, _RB, 64), lambda i: (0, i, 0)),
            pl.BlockSpec((_RB, 64), lambda i: (i, 0)),
            pl.BlockSpec((_RB, 1), lambda i: (i, 0)),
            pl.BlockSpec((1, 64), lambda i: (0, 0)),
            pl.BlockSpec((64, 64), lambda i: (0, 0)),
        ],
        out_specs=[
            pl.BlockSpec((_RB, 32), lambda i: (i, 0)),
            pl.BlockSpec((_RB, 32), lambda i: (i, 0)),
            pl.BlockSpec((_RB, 32), lambda i: (i, 0)),
        ],
        out_shape=[
            jax.ShapeDtypeStruct((_N, 32), jnp.float32),
            jax.ShapeDtypeStruct((_N, 32), jnp.float32),
            jax.ShapeDtypeStruct((_NPAD, 32), jnp.float32),
        ],
    )(acc, u2, dinv, bcat, wcat)


def _tc_dec1(acc, u3, dinv, w2, b2):
    def body(acc_ref, u_ref, dinv_ref, b_ref, w_ref, o_ref):
        t = jnp.dot(acc_ref[0] + acc_ref[1] + u_ref[...], w_ref[...],
                    preferred_element_type=jnp.float32)
        t = dinv_ref[...] * t + b_ref[...]
        o_ref[...] = dinv_ref[...] * jnp.maximum(t, 0.0)

    return pl.pallas_call(
        body,
        grid=(_NPAD // _RB,),
        in_specs=[
            pl.BlockSpec((2, _RB, 32), lambda i: (0, i, 0)),
            pl.BlockSpec((_RB, 32), lambda i: (i, 0)),
            pl.BlockSpec((_RB, 1), lambda i: (i, 0)),
            pl.BlockSpec((1, 64), lambda i: (0, 0)),
            pl.BlockSpec((32, 64), lambda i: (0, 0)),
        ],
        out_specs=pl.BlockSpec((_RB, 64), lambda i: (i, 0)),
        out_shape=jax.ShapeDtypeStruct((_NPAD, 64), jnp.float32),
    )(acc, u3, dinv, b2, w2)


def _tc_dec2(acc, u4, dinv, w3, b3):
    def body(acc_ref, u_ref, dinv_ref, b_ref, w_ref, o_ref):
        t = jnp.dot(acc_ref[0] + acc_ref[1] + u_ref[...], w_ref[...],
                    preferred_element_type=jnp.float32)
        o_ref[...] = 0.5 * jnp.tanh(0.5 * (dinv_ref[...] * t + b_ref[...])) + 0.5

    return pl.pallas_call(
        body,
        grid=(_NPAD // _RB,),
        in_specs=[
            pl.BlockSpec((2, _RB, 64), lambda i: (0, i, 0)),
            pl.BlockSpec((_RB, 64), lambda i: (i, 0)),
            pl.BlockSpec((_RB, 1), lambda i: (i, 0)),
            pl.BlockSpec((1, 128), lambda i: (0, 0)),
            pl.BlockSpec((64, 128), lambda i: (0, 0)),
        ],
        out_specs=pl.BlockSpec((_RB, 128), lambda i: (i, 0)),
        out_shape=jax.ShapeDtypeStruct((_N, 128), jnp.float32),
    )(acc, u4, dinv, b3, w3)


def _tc_adj(z):
    bi, bj = 512, 2048

    def body(a_ref, b_ref, o_ref):
        prod = lax.dot_general(
            a_ref[...], b_ref[...], (((1,), (1,)), ((), ())),
            preferred_element_type=jnp.float32)
        o_ref[...] = 0.5 * jnp.tanh(0.5 * prod) + 0.5

    return pl.pallas_call(
        body,
        grid=(pl.cdiv(_N, bi), pl.cdiv(_N, bj)),
        in_specs=[
            pl.BlockSpec((bi, 32), lambda i, j: (i, 0)),
            pl.BlockSpec((bj, 32), lambda i, j: (j, 0)),
        ],
        out_specs=pl.BlockSpec((bi, bj), lambda i, j: (i, j)),
        out_shape=jax.ShapeDtypeStruct((_N, _N), jnp.float32),
    )(z, z)


def kernel(x_features, edge_index, W1, b1, Wmu, bmu, Wlv, blv, W2, b2, W3, b3):
    ei = edge_index.astype(jnp.int32)
    pad = jnp.full((_EPAD - _E,), _NPAD - 1, jnp.int32)
    s3 = jnp.concatenate([ei[0], pad]).reshape(_NW, _NCH, _CH)
    d3 = jnp.concatenate([ei[1], pad]).reshape(_NW, _NCH, _CH)
    e4 = jnp.stack([s3, d3], axis=2)
    x_pad = jnp.pad(x_features, ((0, _NPAD - _N), (0, 0)))
    wcat = jnp.concatenate([Wmu, Wlv], axis=1)
    bcat = jnp.concatenate([bmu, blv]).reshape(1, 64)

    deg2 = _deg(e4)
    xw1 = _tc_xw1(x_pad, W1)
    y1, dinv = _tc_scale1(deg2, xw1)
    acc1 = _agg64(e4, y1)
    u2 = _tc_elt(acc1, y1, dinv, b1.reshape(1, 64))
    acc2 = _agg64(e4, u2)
    mu, logvar, u3 = _tc_latent(acc2, u2, dinv, bcat, wcat)
    acc3 = _agg32(e4, u3)
    u4 = _tc_dec1(acc3, u3, dinv, W2, b2.reshape(1, 64))
    acc4 = _agg64(e4, u4)
    x_pred = _tc_dec2(acc4, u4, dinv, W3, b3.reshape(1, 128))
    A_pred = _tc_adj(mu)
    z = mu
    return (A_pred, mu, logvar, z, x_pred)


# speedup vs baseline: 1.1020x; 1.1020x over previous
import functools

import jax
import jax.numpy as jnp
from jax import lax
from jax.experimental import pallas as pl
from jax.experimental.pallas import tpu as pltpu
from jax.experimental.pallas import tpu_sc as plsc

_N = 10000
_E = 160000
_NPAD = 10240
_NW = 32
_CH = 128
_NCH = 40
_EPW = _CH * _NCH
_EPAD = _EPW * _NW
_RPW = _NPAD // 16


def _mesh():
    return plsc.VectorSubcoreMesh(core_axis_name="c", subcore_axis_name="s")


def _make_agg(feat):
    @functools.partial(
        pl.kernel,
        mesh=_mesh(),
        compiler_params=pltpu.CompilerParams(use_tc_tiling_on_sc=False),
        out_type=jax.ShapeDtypeStruct((2, _NPAD, feat), jnp.float32),
        scratch_types=[pltpu.VMEM((2, _CH), jnp.int32)] * 8 + [
            pltpu.VMEM((_CH, feat), jnp.float32),
            pltpu.VMEM((_CH, feat), jnp.float32),
            pltpu.VMEM((_CH, feat), jnp.float32),
            pltpu.VMEM((_CH, feat), jnp.float32),
            pltpu.VMEM_SHARED((_NPAD, feat), jnp.float32),
            pltpu.VMEM_SHARED((_NPAD, feat), jnp.float32),
        ] + [pltpu.SemaphoreType.DMA] * 16,
    )
    def agg(e_hbm, tab_hbm, out_hbm,
            i0, i1, i2, i3, i4, i5, i6, i7, r0, r1, r2, r3, tab_sh, acc,
            si0, si1, si2, si3, si4, si5, si6, si7,
            sg0, sg1, sg2, sg3, ss0, ss1, ss2, ss3):
        c = lax.axis_index("c")
        s = lax.axis_index("s")
        wid = s * 2 + c
        idxb = [i0, i1, i2, i3, i4, i5, i6, i7]
        rowsb = [r0, r1, r2, r3]
        si = [si0, si1, si2, si3, si4, si5, si6, si7]
        sg = [sg0, sg1, sg2, sg3]
        ss = [ss0, ss1, ss2, ss3]

        def start_idx(k, b):
            pltpu.async_copy(e_hbm.at[wid, k], idxb[b], si[b])

        def wait_idx(k, b):
            pltpu.make_async_copy(e_hbm.at[wid, k], idxb[b], si[b]).wait()

        def start_gather(b, rb):
            pltpu.async_copy(tab_sh.at[idxb[b].at[0]], rowsb[rb], sg[rb])

        def wait_gather(b, rb):
            pltpu.make_async_copy(
                tab_sh.at[idxb[b].at[0]], rowsb[rb], sg[rb]).wait()

        def start_scat(b, rb):
            pltpu.async_copy(rowsb[rb], acc.at[idxb[b].at[1]], ss[rb],
                             add=True)

        def wait_scat(b, rb):
            pltpu.make_async_copy(
                rowsb[rb], acc.at[idxb[b].at[1]], ss[rb]).wait()

        def step(k, b, rb, has_next, has_prev, has_pref):
            wait_gather(b, rb)
            start_scat(b, rb)
            if has_next:
                if has_prev:
                    wait_scat((b + 7) % 8, (rb + 3) % 4)
                wait_idx(k + 3, (b + 3) % 8)
                start_gather((b + 3) % 8, (rb + 3) % 4)
                if has_pref:
                    start_idx(k + 7, (b + 7) % 8)

        pltpu.sync_copy(tab_hbm.at[pl.ds(s * _RPW, _RPW)],
                        tab_sh.at[pl.ds(s * _RPW, _RPW)])

        def _zero(i, carry):
            for j in range(feat // 16):
                r0[i, pl.ds(j * 16, 16)] = jnp.zeros((16,), jnp.float32)
            return carry

        lax.fori_loop(0, _CH, _zero, 0)
        for t in range(_RPW // _CH):
            pltpu.sync_copy(r0, acc.at[pl.ds(s * _RPW + t * _CH, _CH)])
        for b in range(7):
            start_idx(b, b)
        plsc.subcore_barrier()
        for k in range(3):
            wait_idx(k, k)
            start_gather(k, k)

        step(0, 0, 0, True, False, True)
        for k in range(1, 8):
            step(k, k % 8, k % 4, True, True, k + 7 < _NCH)

        def _mid(t, carry):
            k0 = 8 * t
            for i in range(8):
                step(k0 + i, i, i % 4, True, True, True)
            return carry

        lax.fori_loop(1, _NCH // 8 - 1, _mid, 0)

        for k in range(_NCH - 8, _NCH):
            step(k, k % 8, k % 4, k + 3 < _NCH, k + 3 < _NCH, k + 7 < _NCH)
        for k in range(_NCH - 4, _NCH):
            wait_scat(k % 8, k % 4)

        plsc.subcore_barrier()
        pltpu.sync_copy(acc.at[pl.ds(s * _RPW, _RPW)],
                        out_hbm.at[c, pl.ds(s * _RPW, _RPW)])

    return agg


_agg64 = _make_agg(64)
_agg32 = _make_agg(32)


@functools.partial(
    pl.kernel,
    mesh=_mesh(),
    compiler_params=pltpu.CompilerParams(use_tc_tiling_on_sc=False),
    out_type=jax.ShapeDtypeStruct((2, _NPAD, 16), jnp.float32),
    scratch_types=[
        pltpu.VMEM((2, _CH), jnp.int32),
        pltpu.VMEM((2, _CH), jnp.int32),
        pltpu.VMEM((_CH, 16), jnp.float32),
        pltpu.VMEM_SHARED((_NPAD, 16), jnp.float32),
    ] + [pltpu.SemaphoreType.DMA] * 2,
)
def _deg(e_hbm, out_hbm, i0, i1, rows, acc, si0, si1):
    c = lax.axis_index("c")
    s = lax.axis_index("s")
    wid = s * 2 + c
    idxb = [i0, i1]
    si = [si0, si1]

    def _zero(i, carry):
        rows[i, pl.ds(0, 16)] = jnp.zeros((16,), jnp.float32)
        return carry

    lax.fori_loop(0, _CH, _zero, 0)
    for t in range(_RPW // _CH):
        pltpu.sync_copy(rows, acc.at[pl.ds(s * _RPW + t * _CH, _CH)])

    e0 = jnp.where(lax.iota(jnp.int32, 16) == 0, 1.0, 0.0).astype(jnp.float32)

    def _fill(i, carry):
        rows[i, pl.ds(0, 16)] = e0
        return carry

    lax.fori_loop(0, _CH, _fill, 0)
    pltpu.async_copy(e_hbm.at[wid, 0], i0, si0)
    pltpu.async_copy(e_hbm.at[wid, 1], i1, si1)
    plsc.subcore_barrier()

    def _chunk(j, carry):
        for b in range(2):
            k = 2 * j + b
            pltpu.make_async_copy(e_hbm.at[wid, k], idxb[b], si[b]).wait()
            pltpu.sync_copy(rows, acc.at[idxb[b].at[1]], add=True)
            nxt = k + 2

            @pl.when(nxt < _NCH)
            def _():
                pltpu.async_copy(e_hbm.at[wid, nxt], idxb[b], si[b])

        return carry

    lax.fori_loop(0, _NCH // 2, _chunk, 0)
    plsc.subcore_barrier()
    pltpu.sync_copy(acc.at[pl.ds(s * _RPW, _RPW)],
                    out_hbm.at[c, pl.ds(s * _RPW, _RPW)])


_RB = 1024


def _tc_xw1(x_pad, w1):
    def body(x_ref, w_ref, o_ref):
        o_ref[...] = jnp.dot(x_ref[...], w_ref[...],
                             preferred_element_type=jnp.float32)

    return pl.pallas_call(
        body,
        grid=(_NPAD // _RB,),
        in_specs=[
            pl.BlockSpec((_RB, 128), lambda i: (i, 0)),
            pl.BlockSpec((128, 64), lambda i: (0, 0)),
        ],
        out_specs=pl.BlockSpec((_RB, 64), lambda i: (i, 0)),
        out_shape=jax.ShapeDtypeStruct((_NPAD, 64), jnp.float32),
    )(x_pad, w1)


def _tc_scale1(deg2, xw1):
    def body(deg_ref, xw_ref, y_ref, dinv_ref):
        deg = deg_ref[0, :, 0:1] + deg_ref[1, :, 0:1] + 1.0
        dinv = lax.rsqrt(deg)
        y_ref[...] = dinv * xw_ref[...]
        dinv_ref[...] = dinv

    return pl.pallas_call(
        body,
        grid=(_NPAD // _RB,),
        in_specs=[
            pl.BlockSpec((2, _RB, 16), lambda i: (0, i, 0)),
            pl.BlockSpec((_RB, 64), lambda i: (i, 0)),
        ],
        out_specs=[
            pl.BlockSpec((_RB, 64), lambda i: (i, 0)),
            pl.BlockSpec((_RB, 1), lambda i: (i, 0)),
        ],
        out_shape=[
            jax.ShapeDtypeStruct((_NPAD, 64), jnp.float32),
            jax.ShapeDtypeStruct((_NPAD, 1), jnp.float32),
        ],
    )(deg2, xw1)


def _tc_elt(acc, y1, dinv, b1):
    def body(acc_ref, y_ref, dinv_ref, b_ref, o_ref):
        t = dinv_ref[...] * (acc_ref[0] + acc_ref[1] + y_ref[...]) + b_ref[...]
        o_ref[...] = dinv_ref[...] * jnp.maximum(t, 0.0)

    return pl.pallas_call(
        body,
        grid=(_NPAD // _RB,),
        in_specs=[
            pl.BlockSpec((2, _RB, 64), lambda i: (0, i, 0)),
            pl.BlockSpec((_RB, 64), lambda i: (i, 0)),
            pl.BlockSpec((_RB, 1), lambda i: (i, 0)),
            pl.BlockSpec((1, 64), lambda i: (0, 0)),
        ],
        out_specs=pl.BlockSpec((_RB, 64), lambda i: (i, 0)),
        out_shape=jax.ShapeDtypeStruct((_NPAD, 64), jnp.float32),
    )(acc, y1, dinv, b1)


def _tc_latent(acc, u2, dinv, bcat, wcat):
    def body(acc_ref, u_ref, dinv_ref, b_ref, w_ref, mu_ref, lv_ref, u3_ref):
        t = jnp.dot(acc_ref[0] + acc_ref[1] + u_ref[...], w_ref[...],
                    preferred_element_type=jnp.float32)
        outcat = dinv_ref[...] * t + b_ref[...]
        mu = outcat[:, 0:32]
        mu_ref[...] = mu
        lv_ref[...] = outcat[:, 32:64]
        u3_ref[...] = dinv_ref[...] * mu

    return pl.pallas_call(
        body,
        grid=(_NPAD // _RB,),
        in_specs=[
            pl.BlockSpec((2, _RB, 64), lambda i: (0, i, 0)),
            pl.BlockSpec((_RB, 64), lambda i: (i, 0)),
            pl.BlockSpec((_RB, 1), lambda i: (i, 0)),
            pl.BlockSpec((1, 64), lambda i: (0, 0)),
            pl.BlockSpec((64, 64), lambda i: (0, 0)),
        ],
        out_specs=[
            pl.BlockSpec((_RB, 32), lambda i: (i, 0)),
            pl.BlockSpec((_RB, 32), lambda i: (i, 0)),
            pl.BlockSpec((_RB, 32), lambda i: (i, 0)),
        ],
        out_shape=[
            jax.ShapeDtypeStruct((_N, 32), jnp.float32),
            jax.ShapeDtypeStruct((_N, 32), jnp.float32),
            jax.ShapeDtypeStruct((_NPAD, 32), jnp.float32),
        ],
    )(acc, u2, dinv, bcat, wcat)


def _tc_dec1(acc, u3, dinv, w2, b2):
    def body(acc_ref, u_ref, dinv_ref, b_ref, w_ref, o_ref):
        t = jnp.dot(acc_ref[0] + acc_ref[1] + u_ref[...], w_ref[...],
                    preferred_element_type=jnp.float32)
        t = dinv_ref[...] * t + b_ref[...]
        o_ref[...] = dinv_ref[...] * jnp.maximum(t, 0.0)

    return pl.pallas_call(
        body,
        grid=(_NPAD // _RB,),
        in_specs=[
            pl.BlockSpec((2, _RB, 32), lambda i: (0, i, 0)),
            pl.BlockSpec((_RB, 32), lambda i: (i, 0)),
            pl.BlockSpec((_RB, 1), lambda i: (i, 0)),
            pl.BlockSpec((1, 64), lambda i: (0, 0)),
            pl.BlockSpec((32, 64), lambda i: (0, 0)),
        ],
        out_specs=pl.BlockSpec((_RB, 64), lambda i: (i, 0)),
        out_shape=jax.ShapeDtypeStruct((_NPAD, 64), jnp.float32),
    )(acc, u3, dinv, b2, w2)


def _tc_dec2(acc, u4, dinv, w3, b3):
    def body(acc_ref, u_ref, dinv_ref, b_ref, w_ref, o_ref):
        t = jnp.dot(acc_ref[0] + acc_ref[1] + u_ref[...], w_ref[...],
                    preferred_element_type=jnp.float32)
        o_ref[...] = 0.5 * jnp.tanh(0.5 * (dinv_ref[...] * t + b_ref[...])) + 0.5

    return pl.pallas_call(
        body,
        grid=(_NPAD // _RB,),
        in_specs=[
            pl.BlockSpec((2, _RB, 64), lambda i: (0, i, 0)),
            pl.BlockSpec((_RB, 64), lambda i: (i, 0)),
            pl.BlockSpec((_RB, 1), lambda i: (i, 0)),
            pl.BlockSpec((1, 128), lambda i: (0, 0)),
            pl.BlockSpec((64, 128), lambda i: (0, 0)),
        ],
        out_specs=pl.BlockSpec((_RB, 128), lambda i: (i, 0)),
        out_shape=jax.ShapeDtypeStruct((_N, 128), jnp.float32),
    )(acc, u4, dinv, b3, w3)


def _tc_adj(z):
    bi, bj = 2048, 1024

    def body(a_ref, b_ref, o_ref):
        prod = lax.dot_general(
            a_ref[...], b_ref[...], (((1,), (1,)), ((), ())),
            preferred_element_type=jnp.float32)
        o_ref[...] = 0.5 * jnp.tanh(0.5 * prod) + 0.5

    return pl.pallas_call(
        body,
        grid=(pl.cdiv(_N, bi), pl.cdiv(_N, bj)),
        in_specs=[
            pl.BlockSpec((bi, 32), lambda i, j: (i, 0)),
            pl.BlockSpec((bj, 32), lambda i, j: (j, 0)),
        ],
        out_specs=pl.BlockSpec((bi, bj), lambda i, j: (i, j)),
        out_shape=jax.ShapeDtypeStruct((_N, _N), jnp.float32),
    )(z, z)


def kernel(x_features, edge_index, W1, b1, Wmu, bmu, Wlv, blv, W2, b2, W3, b3):
    ei = edge_index.astype(jnp.int32)
    pad = jnp.full((_EPAD - _E,), _NPAD - 1, jnp.int32)
    s3 = jnp.concatenate([ei[0], pad]).reshape(_NW, _NCH, _CH)
    d3 = jnp.concatenate([ei[1], pad]).reshape(_NW, _NCH, _CH)
    e4 = jnp.stack([s3, d3], axis=2)
    x_pad = jnp.pad(x_features, ((0, _NPAD - _N), (0, 0)))
    wcat = jnp.concatenate([Wmu, Wlv], axis=1)
    bcat = jnp.concatenate([bmu, blv]).reshape(1, 64)

    deg2 = _deg(e4)
    xw1 = _tc_xw1(x_pad, W1)
    y1, dinv = _tc_scale1(deg2, xw1)
    acc1 = _agg64(e4, y1)
    u2 = _tc_elt(acc1, y1, dinv, b1.reshape(1, 64))
    acc2 = _agg64(e4, u2)
    mu, logvar, u3 = _tc_latent(acc2, u2, dinv, bcat, wcat)
    acc3 = _agg32(e4, u3)
    u4 = _tc_dec1(acc3, u3, dinv, W2, b2.reshape(1, 64))
    acc4 = _agg64(e4, u4)
    x_pred = _tc_dec2(acc4, u4, dinv, W3, b3.reshape(1, 128))
    A_pred = _tc_adj(mu)
    z = mu
    return (A_pred, mu, logvar, z, x_pred)


# speedup vs baseline: 1.1083x; 1.0058x over previous
import functools

import jax
import jax.numpy as jnp
from jax import lax
from jax.experimental import pallas as pl
from jax.experimental.pallas import tpu as pltpu
from jax.experimental.pallas import tpu_sc as plsc

_N = 10000
_E = 160000
_NPAD = 10240
_NW = 32
_CH = 128
_NCH = 40
_EPW = _CH * _NCH
_EPAD = _EPW * _NW
_RPW = _NPAD // 16


def _mesh():
    return plsc.VectorSubcoreMesh(core_axis_name="c", subcore_axis_name="s")


def _make_agg(feat):
    @functools.partial(
        pl.kernel,
        mesh=_mesh(),
        compiler_params=pltpu.CompilerParams(use_tc_tiling_on_sc=False),
        out_type=jax.ShapeDtypeStruct((2, _NPAD, feat), jnp.float32),
        scratch_types=[pltpu.VMEM((2, _CH), jnp.int32)] * 8 + [
            pltpu.VMEM((_CH, feat), jnp.float32),
            pltpu.VMEM((_CH, feat), jnp.float32),
            pltpu.VMEM((_CH, feat), jnp.float32),
            pltpu.VMEM((_CH, feat), jnp.float32),
            pltpu.VMEM_SHARED((_NPAD, feat), jnp.float32),
            pltpu.VMEM_SHARED((_NPAD, feat), jnp.float32),
        ] + [pltpu.SemaphoreType.DMA] * 16,
    )
    def agg(e_hbm, tab_hbm, out_hbm,
            i0, i1, i2, i3, i4, i5, i6, i7, r0, r1, r2, r3, tab_sh, acc,
            si0, si1, si2, si3, si4, si5, si6, si7,
            sg0, sg1, sg2, sg3, ss0, ss1, ss2, ss3):
        c = lax.axis_index("c")
        s = lax.axis_index("s")
        wid = s * 2 + c
        idxb = [i0, i1, i2, i3, i4, i5, i6, i7]
        rowsb = [r0, r1, r2, r3]
        si = [si0, si1, si2, si3, si4, si5, si6, si7]
        sg = [sg0, sg1, sg2, sg3]
        ss = [ss0, ss1, ss2, ss3]

        def start_idx(k, b):
            pltpu.async_copy(e_hbm.at[wid, k], idxb[b], si[b])

        def wait_idx(k, b):
            pltpu.make_async_copy(e_hbm.at[wid, k], idxb[b], si[b]).wait()

        def start_gather(b, rb):
            pltpu.async_copy(tab_sh.at[idxb[b].at[0]], rowsb[rb], sg[rb])

        def wait_gather(b, rb):
            pltpu.make_async_copy(
                tab_sh.at[idxb[b].at[0]], rowsb[rb], sg[rb]).wait()

        def start_scat(b, rb):
            pltpu.async_copy(rowsb[rb], acc.at[idxb[b].at[1]], ss[rb],
                             add=True)

        def wait_scat(b, rb):
            pltpu.make_async_copy(
                rowsb[rb], acc.at[idxb[b].at[1]], ss[rb]).wait()

        def step(k, b, rb, has_next, has_prev, has_pref):
            wait_gather(b, rb)
            start_scat(b, rb)
            if has_next:
                if has_prev:
                    wait_scat((b + 7) % 8, (rb + 3) % 4)
                wait_idx(k + 3, (b + 3) % 8)
                start_gather((b + 3) % 8, (rb + 3) % 4)
                if has_pref:
                    start_idx(k + 7, (b + 7) % 8)

        pltpu.sync_copy(tab_hbm.at[pl.ds(s * _RPW, _RPW)],
                        tab_sh.at[pl.ds(s * _RPW, _RPW)])

        def _zero(i, carry):
            for j in range(feat // 16):
                r0[i, pl.ds(j * 16, 16)] = jnp.zeros((16,), jnp.float32)
            return carry

        lax.fori_loop(0, _CH, _zero, 0)
        for t in range(_RPW // _CH):
            pltpu.sync_copy(r0, acc.at[pl.ds(s * _RPW + t * _CH, _CH)])
        for b in range(7):
            start_idx(b, b)
        plsc.subcore_barrier()
        for k in range(3):
            wait_idx(k, k)
            start_gather(k, k)

        step(0, 0, 0, True, False, True)
        for k in range(1, 8):
            step(k, k % 8, k % 4, True, True, k + 7 < _NCH)

        def _mid(t, carry):
            k0 = 8 * t
            for i in range(8):
                step(k0 + i, i, i % 4, True, True, True)
            return carry

        lax.fori_loop(1, _NCH // 8 - 1, _mid, 0)

        for k in range(_NCH - 8, _NCH):
            step(k, k % 8, k % 4, k + 3 < _NCH, k + 3 < _NCH, k + 7 < _NCH)
        for k in range(_NCH - 4, _NCH):
            wait_scat(k % 8, k % 4)

        plsc.subcore_barrier()
        pltpu.sync_copy(acc.at[pl.ds(s * _RPW, _RPW)],
                        out_hbm.at[c, pl.ds(s * _RPW, _RPW)])

    return agg


_agg64 = _make_agg(64)
_agg32 = _make_agg(32)


@functools.partial(
    pl.kernel,
    mesh=_mesh(),
    compiler_params=pltpu.CompilerParams(use_tc_tiling_on_sc=False),
    out_type=jax.ShapeDtypeStruct((2, _NPAD, 16), jnp.float32),
    scratch_types=[
        pltpu.VMEM((2, _CH), jnp.int32),
        pltpu.VMEM((2, _CH), jnp.int32),
        pltpu.VMEM((_CH, 16), jnp.float32),
        pltpu.VMEM_SHARED((_NPAD, 16), jnp.float32),
    ] + [pltpu.SemaphoreType.DMA] * 2,
)
def _deg(e_hbm, out_hbm, i0, i1, rows, acc, si0, si1):
    c = lax.axis_index("c")
    s = lax.axis_index("s")
    wid = s * 2 + c
    idxb = [i0, i1]
    si = [si0, si1]

    def _zero(i, carry):
        rows[i, pl.ds(0, 16)] = jnp.zeros((16,), jnp.float32)
        return carry

    lax.fori_loop(0, _CH, _zero, 0)
    for t in range(_RPW // _CH):
        pltpu.sync_copy(rows, acc.at[pl.ds(s * _RPW + t * _CH, _CH)])

    e0 = jnp.where(lax.iota(jnp.int32, 16) == 0, 1.0, 0.0).astype(jnp.float32)

    def _fill(i, carry):
        rows[i, pl.ds(0, 16)] = e0
        return carry

    lax.fori_loop(0, _CH, _fill, 0)
    pltpu.async_copy(e_hbm.at[wid, 0], i0, si0)
    pltpu.async_copy(e_hbm.at[wid, 1], i1, si1)
    plsc.subcore_barrier()

    def _chunk(j, carry):
        for b in range(2):
            k = 2 * j + b
            pltpu.make_async_copy(e_hbm.at[wid, k], idxb[b], si[b]).wait()
            pltpu.sync_copy(rows, acc.at[idxb[b].at[1]], add=True)
            nxt = k + 2

            @pl.when(nxt < _NCH)
            def _():
                pltpu.async_copy(e_hbm.at[wid, nxt], idxb[b], si[b])

        return carry

    lax.fori_loop(0, _NCH // 2, _chunk, 0)
    plsc.subcore_barrier()
    pltpu.sync_copy(acc.at[pl.ds(s * _RPW, _RPW)],
                    out_hbm.at[c, pl.ds(s * _RPW, _RPW)])


_RB = 1024


def _tc_xw1(x_pad, w1):
    def body(x_ref, w_ref, o_ref):
        o_ref[...] = jnp.dot(x_ref[...], w_ref[...],
                             preferred_element_type=jnp.float32)

    return pl.pallas_call(
        body,
        grid=(_NPAD // _RB,),
        in_specs=[
            pl.BlockSpec((_RB, 128), lambda i: (i, 0)),
            pl.BlockSpec((128, 64), lambda i: (0, 0)),
        ],
        out_specs=pl.BlockSpec((_RB, 64), lambda i: (i, 0)),
        out_shape=jax.ShapeDtypeStruct((_NPAD, 64), jnp.float32),
    )(x_pad, w1)


def _tc_scale1(deg2, xw1):
    def body(deg_ref, xw_ref, y_ref, dinv_ref):
        deg = deg_ref[0, :, 0:1] + deg_ref[1, :, 0:1] + 1.0
        dinv = lax.rsqrt(deg)
        y_ref[...] = dinv * xw_ref[...]
        dinv_ref[...] = dinv

    return pl.pallas_call(
        body,
        grid=(_NPAD // _RB,),
        in_specs=[
            pl.BlockSpec((2, _RB, 16), lambda i: (0, i, 0)),
            pl.BlockSpec((_RB, 64), lambda i: (i, 0)),
        ],
        out_specs=[
            pl.BlockSpec((_RB, 64), lambda i: (i, 0)),
            pl.BlockSpec((_RB, 1), lambda i: (i, 0)),
        ],
        out_shape=[
            jax.ShapeDtypeStruct((_NPAD, 64), jnp.float32),
            jax.ShapeDtypeStruct((_NPAD, 1), jnp.float32),
        ],
    )(deg2, xw1)


def _tc_elt(acc, y1, dinv, b1):
    def body(acc_ref, y_ref, dinv_ref, b_ref, o_ref):
        t = dinv_ref[...] * (acc_ref[0] + acc_ref[1] + y_ref[...]) + b_ref[...]
        o_ref[...] = dinv_ref[...] * jnp.maximum(t, 0.0)

    return pl.pallas_call(
        body,
        grid=(_NPAD // _RB,),
        in_specs=[
            pl.BlockSpec((2, _RB, 64), lambda i: (0, i, 0)),
            pl.BlockSpec((_RB, 64), lambda i: (i, 0)),
            pl.BlockSpec((_RB, 1), lambda i: (i, 0)),
            pl.BlockSpec((1, 64), lambda i: (0, 0)),
        ],
        out_specs=pl.BlockSpec((_RB, 64), lambda i: (i, 0)),
        out_shape=jax.ShapeDtypeStruct((_NPAD, 64), jnp.float32),
    )(acc, y1, dinv, b1)


def _tc_latent(acc, u2, dinv, bcat, wcat):
    def body(acc_ref, u_ref, dinv_ref, b_ref, w_ref, mu_ref, lv_ref, u3_ref):
        t = jnp.dot(acc_ref[0] + acc_ref[1] + u_ref[...], w_ref[...],
                    preferred_element_type=jnp.float32)
        outcat = dinv_ref[...] * t + b_ref[...]
        mu = outcat[:, 0:32]
        mu_ref[...] = mu
        lv_ref[...] = outcat[:, 32:64]
        u3_ref[...] = dinv_ref[...] * mu

    return pl.pallas_call(
        body,
        grid=(_NPAD // _RB,),
        in_specs=[
            pl.BlockSpec((2, _RB, 64), lambda i: (0, i, 0)),
            pl.BlockSpec((_RB, 64), lambda i: (i, 0)),
            pl.BlockSpec((_RB, 1), lambda i: (i, 0)),
            pl.BlockSpec((1, 64), lambda i: (0, 0)),
            pl.BlockSpec((64, 64), lambda i: (0, 0)),
        ],
        out_specs=[
            pl.BlockSpec((_RB, 32), lambda i: (i, 0)),
            pl.BlockSpec((_RB, 32), lambda i: (i, 0)),
            pl.BlockSpec((_RB, 32), lambda i: (i, 0)),
        ],
        out_shape=[
            jax.ShapeDtypeStruct((_N, 32), jnp.float32),
            jax.ShapeDtypeStruct((_N, 32), jnp.float32),
            jax.ShapeDtypeStruct((_NPAD, 32), jnp.float32),
        ],
    )(acc, u2, dinv, bcat, wcat)


def _tc_dec1(acc, u3, dinv, w2, b2):
    def body(acc_ref, u_ref, dinv_ref, b_ref, w_ref, o_ref):
        t = jnp.dot(acc_ref[0] + acc_ref[1] + u_ref[...], w_ref[...],
                    preferred_element_type=jnp.float32)
        t = dinv_ref[...] * t + b_ref[...]
        o_ref[...] = dinv_ref[...] * jnp.maximum(t, 0.0)

    return pl.pallas_call(
        body,
        grid=(_NPAD // _RB,),
        in_specs=[
            pl.BlockSpec((2, _RB, 32), lambda i: (0, i, 0)),
            pl.BlockSpec((_RB, 32), lambda i: (i, 0)),
            pl.BlockSpec((_RB, 1), lambda i: (i, 0)),
            pl.BlockSpec((1, 64), lambda i: (0, 0)),
            pl.BlockSpec((32, 64), lambda i: (0, 0)),
        ],
        out_specs=pl.BlockSpec((_RB, 64), lambda i: (i, 0)),
        out_shape=jax.ShapeDtypeStruct((_NPAD, 64), jnp.float32),
    )(acc, u3, dinv, b2, w2)


def _tc_dec2(acc, u4, dinv, w3, b3):
    def body(acc_ref, u_ref, dinv_ref, b_ref, w_ref, o_ref):
        t = jnp.dot(acc_ref[0] + acc_ref[1] + u_ref[...], w_ref[...],
                    preferred_element_type=jnp.float32)
        o_ref[...] = 0.5 * jnp.tanh(0.5 * (dinv_ref[...] * t + b_ref[...])) + 0.5

    return pl.pallas_call(
        body,
        grid=(_NPAD // _RB,),
        in_specs=[
            pl.BlockSpec((2, _RB, 64), lambda i: (0, i, 0)),
            pl.BlockSpec((_RB, 64), lambda i: (i, 0)),
            pl.BlockSpec((_RB, 1), lambda i: (i, 0)),
            pl.BlockSpec((1, 128), lambda i: (0, 0)),
            pl.BlockSpec((64, 128), lambda i: (0, 0)),
        ],
        out_specs=pl.BlockSpec((_RB, 128), lambda i: (i, 0)),
        out_shape=jax.ShapeDtypeStruct((_N, 128), jnp.float32),
    )(acc, u4, dinv, b3, w3)


def _tc_adj(z):
    bi, bj = 2048, 2048

    def body(a_ref, b_ref, o_ref):
        prod = lax.dot_general(
            a_ref[...], b_ref[...], (((1,), (1,)), ((), ())),
            preferred_element_type=jnp.float32)
        o_ref[...] = 0.5 * jnp.tanh(0.5 * prod) + 0.5

    return pl.pallas_call(
        body,
        grid=(pl.cdiv(_N, bi), pl.cdiv(_N, bj)),
        in_specs=[
            pl.BlockSpec((bi, 32), lambda i, j: (i, 0)),
            pl.BlockSpec((bj, 32), lambda i, j: (j, 0)),
        ],
        out_specs=pl.BlockSpec((bi, bj), lambda i, j: (i, j)),
        out_shape=jax.ShapeDtypeStruct((_N, _N), jnp.float32),
    )(z, z)


def kernel(x_features, edge_index, W1, b1, Wmu, bmu, Wlv, blv, W2, b2, W3, b3):
    ei = edge_index.astype(jnp.int32)
    pad = jnp.full((_EPAD - _E,), _NPAD - 1, jnp.int32)
    s3 = jnp.concatenate([ei[0], pad]).reshape(_NW, _NCH, _CH)
    d3 = jnp.concatenate([ei[1], pad]).reshape(_NW, _NCH, _CH)
    e4 = jnp.stack([s3, d3], axis=2)
    x_pad = jnp.pad(x_features, ((0, _NPAD - _N), (0, 0)))
    wcat = jnp.concatenate([Wmu, Wlv], axis=1)
    bcat = jnp.concatenate([bmu, blv]).reshape(1, 64)

    deg2 = _deg(e4)
    xw1 = _tc_xw1(x_pad, W1)
    y1, dinv = _tc_scale1(deg2, xw1)
    acc1 = _agg64(e4, y1)
    u2 = _tc_elt(acc1, y1, dinv, b1.reshape(1, 64))
    acc2 = _agg64(e4, u2)
    mu, logvar, u3 = _tc_latent(acc2, u2, dinv, bcat, wcat)
    acc3 = _agg32(e4, u3)
    u4 = _tc_dec1(acc3, u3, dinv, W2, b2.reshape(1, 64))
    acc4 = _agg64(e4, u4)
    x_pred = _tc_dec2(acc4, u4, dinv, W3, b3.reshape(1, 128))
    A_pred = _tc_adj(mu)
    z = mu
    return (A_pred, mu, logvar, z, x_pred)


# speedup vs baseline: 1.1148x; 1.0059x over previous
import functools

import jax
import jax.numpy as jnp
from jax import lax
from jax.experimental import pallas as pl
from jax.experimental.pallas import tpu as pltpu
from jax.experimental.pallas import tpu_sc as plsc

_N = 10000
_E = 160000
_NPAD = 10240
_NW = 32
_CH = 128
_NCH = 40
_EPW = _CH * _NCH
_EPAD = _EPW * _NW
_RPW = _NPAD // 16


def _mesh():
    return plsc.VectorSubcoreMesh(core_axis_name="c", subcore_axis_name="s")


def _make_agg(feat):
    @functools.partial(
        pl.kernel,
        mesh=_mesh(),
        compiler_params=pltpu.CompilerParams(use_tc_tiling_on_sc=False),
        out_type=jax.ShapeDtypeStruct((2, _NPAD, feat), jnp.float32),
        scratch_types=[pltpu.VMEM((2, _CH), jnp.int32)] * 8 + [
            pltpu.VMEM((_CH, feat), jnp.float32),
            pltpu.VMEM((_CH, feat), jnp.float32),
            pltpu.VMEM((_CH, feat), jnp.float32),
            pltpu.VMEM((_CH, feat), jnp.float32),
            pltpu.VMEM_SHARED((_NPAD, feat), jnp.float32),
            pltpu.VMEM_SHARED((_NPAD, feat), jnp.float32),
        ] + [pltpu.SemaphoreType.DMA] * 16,
    )
    def agg(e_hbm, tab_hbm, out_hbm,
            i0, i1, i2, i3, i4, i5, i6, i7, r0, r1, r2, r3, tab_sh, acc,
            si0, si1, si2, si3, si4, si5, si6, si7,
            sg0, sg1, sg2, sg3, ss0, ss1, ss2, ss3):
        c = lax.axis_index("c")
        s = lax.axis_index("s")
        wid = s * 2 + c
        idxb = [i0, i1, i2, i3, i4, i5, i6, i7]
        rowsb = [r0, r1, r2, r3]
        si = [si0, si1, si2, si3, si4, si5, si6, si7]
        sg = [sg0, sg1, sg2, sg3]
        ss = [ss0, ss1, ss2, ss3]

        def start_idx(k, b):
            pltpu.async_copy(e_hbm.at[wid, k], idxb[b], si[b])

        def wait_idx(k, b):
            pltpu.make_async_copy(e_hbm.at[wid, k], idxb[b], si[b]).wait()

        def start_gather(b, rb):
            pltpu.async_copy(tab_sh.at[idxb[b].at[0]], rowsb[rb], sg[rb])

        def wait_gather(b, rb):
            pltpu.make_async_copy(
                tab_sh.at[idxb[b].at[0]], rowsb[rb], sg[rb]).wait()

        def start_scat(b, rb):
            pltpu.async_copy(rowsb[rb], acc.at[idxb[b].at[1]], ss[rb],
                             add=True)

        def wait_scat(b, rb):
            pltpu.make_async_copy(
                rowsb[rb], acc.at[idxb[b].at[1]], ss[rb]).wait()

        def step(k, b, rb, has_next, has_prev, has_pref):
            wait_gather(b, rb)
            start_scat(b, rb)
            if has_next:
                if has_prev:
                    wait_scat((b + 7) % 8, (rb + 3) % 4)
                wait_idx(k + 3, (b + 3) % 8)
                start_gather((b + 3) % 8, (rb + 3) % 4)
                if has_pref:
                    start_idx(k + 7, (b + 7) % 8)

        pltpu.sync_copy(tab_hbm.at[pl.ds(s * _RPW, _RPW)],
                        tab_sh.at[pl.ds(s * _RPW, _RPW)])

        def _zero(i, carry):
            for j in range(feat // 16):
                r0[i, pl.ds(j * 16, 16)] = jnp.zeros((16,), jnp.float32)
            return carry

        lax.fori_loop(0, _CH, _zero, 0)
        for t in range(_RPW // _CH):
            pltpu.sync_copy(r0, acc.at[pl.ds(s * _RPW + t * _CH, _CH)])
        for b in range(7):
            start_idx(b, b)
        plsc.subcore_barrier()
        for k in range(3):
            wait_idx(k, k)
            start_gather(k, k)

        step(0, 0, 0, True, False, True)
        for k in range(1, 8):
            step(k, k % 8, k % 4, True, True, k + 7 < _NCH)

        def _mid(t, carry):
            k0 = 8 * t
            for i in range(8):
                step(k0 + i, i, i % 4, True, True, True)
            return carry

        lax.fori_loop(1, _NCH // 8 - 1, _mid, 0)

        for k in range(_NCH - 8, _NCH):
            step(k, k % 8, k % 4, k + 3 < _NCH, k + 3 < _NCH, k + 7 < _NCH)
        for k in range(_NCH - 4, _NCH):
            wait_scat(k % 8, k % 4)

        plsc.subcore_barrier()
        pltpu.sync_copy(acc.at[pl.ds(s * _RPW, _RPW)],
                        out_hbm.at[c, pl.ds(s * _RPW, _RPW)])

    return agg


_agg64 = _make_agg(64)
_agg32 = _make_agg(32)


@functools.partial(
    pl.kernel,
    mesh=_mesh(),
    compiler_params=pltpu.CompilerParams(use_tc_tiling_on_sc=False),
    out_type=jax.ShapeDtypeStruct((2, _NPAD, 16), jnp.float32),
    scratch_types=[
        pltpu.VMEM((2, _CH), jnp.int32),
        pltpu.VMEM((2, _CH), jnp.int32),
        pltpu.VMEM((_CH, 16), jnp.float32),
        pltpu.VMEM_SHARED((_NPAD, 16), jnp.float32),
    ] + [pltpu.SemaphoreType.DMA] * 2,
)
def _deg(e_hbm, out_hbm, i0, i1, rows, acc, si0, si1):
    c = lax.axis_index("c")
    s = lax.axis_index("s")
    wid = s * 2 + c
    idxb = [i0, i1]
    si = [si0, si1]

    def _zero(i, carry):
        rows[i, pl.ds(0, 16)] = jnp.zeros((16,), jnp.float32)
        return carry

    lax.fori_loop(0, _CH, _zero, 0)
    for t in range(_RPW // _CH):
        pltpu.sync_copy(rows, acc.at[pl.ds(s * _RPW + t * _CH, _CH)])

    e0 = jnp.where(lax.iota(jnp.int32, 16) == 0, 1.0, 0.0).astype(jnp.float32)

    def _fill(i, carry):
        rows[i, pl.ds(0, 16)] = e0
        return carry

    lax.fori_loop(0, _CH, _fill, 0)
    pltpu.async_copy(e_hbm.at[wid, 0], i0, si0)
    pltpu.async_copy(e_hbm.at[wid, 1], i1, si1)
    plsc.subcore_barrier()

    def _chunk(j, carry):
        for b in range(2):
            k = 2 * j + b
            pltpu.make_async_copy(e_hbm.at[wid, k], idxb[b], si[b]).wait()
            pltpu.sync_copy(rows, acc.at[idxb[b].at[1]], add=True)
            nxt = k + 2

            @pl.when(nxt < _NCH)
            def _():
                pltpu.async_copy(e_hbm.at[wid, nxt], idxb[b], si[b])

        return carry

    lax.fori_loop(0, _NCH // 2, _chunk, 0)
    plsc.subcore_barrier()
    pltpu.sync_copy(acc.at[pl.ds(s * _RPW, _RPW)],
                    out_hbm.at[c, pl.ds(s * _RPW, _RPW)])


_RB = 1024


def _tc_xw1(x_pad, w1):
    def body(x_ref, w_ref, o_ref):
        o_ref[...] = jnp.dot(x_ref[...], w_ref[...],
                             preferred_element_type=jnp.float32)

    return pl.pallas_call(
        body,
        grid=(_NPAD // _RB,),
        in_specs=[
            pl.BlockSpec((_RB, 128), lambda i: (i, 0)),
            pl.BlockSpec((128, 64), lambda i: (0, 0)),
        ],
        out_specs=pl.BlockSpec((_RB, 64), lambda i: (i, 0)),
        out_shape=jax.ShapeDtypeStruct((_NPAD, 64), jnp.float32),
    )(x_pad, w1)


def _tc_scale1(deg2, xw1):
    def body(deg_ref, xw_ref, y_ref, dinv_ref):
        deg = deg_ref[0, :, 0:1] + deg_ref[1, :, 0:1] + 1.0
        dinv = lax.rsqrt(deg)
        y_ref[...] = dinv * xw_ref[...]
        dinv_ref[...] = dinv

    return pl.pallas_call(
        body,
        grid=(_NPAD // _RB,),
        in_specs=[
            pl.BlockSpec((2, _RB, 16), lambda i: (0, i, 0)),
            pl.BlockSpec((_RB, 64), lambda i: (i, 0)),
        ],
        out_specs=[
            pl.BlockSpec((_RB, 64), lambda i: (i, 0)),
            pl.BlockSpec((_RB, 1), lambda i: (i, 0)),
        ],
        out_shape=[
            jax.ShapeDtypeStruct((_NPAD, 64), jnp.float32),
            jax.ShapeDtypeStruct((_NPAD, 1), jnp.float32),
        ],
    )(deg2, xw1)


def _tc_elt(acc, y1, dinv, b1):
    def body(acc_ref, y_ref, dinv_ref, b_ref, o_ref):
        t = dinv_ref[...] * (acc_ref[0] + acc_ref[1] + y_ref[...]) + b_ref[...]
        o_ref[...] = dinv_ref[...] * jnp.maximum(t, 0.0)

    return pl.pallas_call(
        body,
        grid=(_NPAD // _RB,),
        in_specs=[
            pl.BlockSpec((2, _RB, 64), lambda i: (0, i, 0)),
            pl.BlockSpec((_RB, 64), lambda i: (i, 0)),
            pl.BlockSpec((_RB, 1), lambda i: (i, 0)),
            pl.BlockSpec((1, 64), lambda i: (0, 0)),
        ],
        out_specs=pl.BlockSpec((_RB, 64), lambda i: (i, 0)),
        out_shape=jax.ShapeDtypeStruct((_NPAD, 64), jnp.float32),
    )(acc, y1, dinv, b1)


def _tc_latent(acc, u2, dinv, bcat, wcat):
    def body(acc_ref, u_ref, dinv_ref, b_ref, w_ref, mu_ref, lv_ref, u3_ref):
        t = jnp.dot(acc_ref[0] + acc_ref[1] + u_ref[...], w_ref[...],
                    preferred_element_type=jnp.float32)
        outcat = dinv_ref[...] * t + b_ref[...]
        mu = outcat[:, 0:32]
        mu_ref[...] = mu
        lv_ref[...] = outcat[:, 32:64]
        u3_ref[...] = dinv_ref[...] * mu

    return pl.pallas_call(
        body,
        grid=(_NPAD // _RB,),
        in_specs=[
            pl.BlockSpec((2, _RB, 64), lambda i: (0, i, 0)),
            pl.BlockSpec((_RB, 64), lambda i: (i, 0)),
            pl.BlockSpec((_RB, 1), lambda i: (i, 0)),
            pl.BlockSpec((1, 64), lambda i: (0, 0)),
            pl.BlockSpec((64, 64), lambda i: (0, 0)),
        ],
        out_specs=[
            pl.BlockSpec((_RB, 32), lambda i: (i, 0)),
            pl.BlockSpec((_RB, 32), lambda i: (i, 0)),
            pl.BlockSpec((_RB, 32), lambda i: (i, 0)),
        ],
        out_shape=[
            jax.ShapeDtypeStruct((_N, 32), jnp.float32),
            jax.ShapeDtypeStruct((_N, 32), jnp.float32),
            jax.ShapeDtypeStruct((_NPAD, 32), jnp.float32),
        ],
    )(acc, u2, dinv, bcat, wcat)


def _tc_dec1(acc, u3, dinv, w2, b2):
    def body(acc_ref, u_ref, dinv_ref, b_ref, w_ref, o_ref):
        t = jnp.dot(acc_ref[0] + acc_ref[1] + u_ref[...], w_ref[...],
                    preferred_element_type=jnp.float32)
        t = dinv_ref[...] * t + b_ref[...]
        o_ref[...] = dinv_ref[...] * jnp.maximum(t, 0.0)

    return pl.pallas_call(
        body,
        grid=(_NPAD // _RB,),
        in_specs=[
            pl.BlockSpec((2, _RB, 32), lambda i: (0, i, 0)),
            pl.BlockSpec((_RB, 32), lambda i: (i, 0)),
            pl.BlockSpec((_RB, 1), lambda i: (i, 0)),
            pl.BlockSpec((1, 64), lambda i: (0, 0)),
            pl.BlockSpec((32, 64), lambda i: (0, 0)),
        ],
        out_specs=pl.BlockSpec((_RB, 64), lambda i: (i, 0)),
        out_shape=jax.ShapeDtypeStruct((_NPAD, 64), jnp.float32),
    )(acc, u3, dinv, b2, w2)


def _tc_dec2(acc, u4, dinv, w3, b3):
    def body(acc_ref, u_ref, dinv_ref, b_ref, w_ref, o_ref):
        t = jnp.dot(acc_ref[0] + acc_ref[1] + u_ref[...], w_ref[...],
                    preferred_element_type=jnp.float32)
        o_ref[...] = 0.5 * jnp.tanh(0.5 * (dinv_ref[...] * t + b_ref[...])) + 0.5

    return pl.pallas_call(
        body,
        grid=(_NPAD // _RB,),
        in_specs=[
            pl.BlockSpec((2, _RB, 64), lambda i: (0, i, 0)),
            pl.BlockSpec((_RB, 64), lambda i: (i, 0)),
            pl.BlockSpec((_RB, 1), lambda i: (i, 0)),
            pl.BlockSpec((1, 128), lambda i: (0, 0)),
            pl.BlockSpec((64, 128), lambda i: (0, 0)),
        ],
        out_specs=pl.BlockSpec((_RB, 128), lambda i: (i, 0)),
        out_shape=jax.ShapeDtypeStruct((_N, 128), jnp.float32),
    )(acc, u4, dinv, b3, w3)


def _tc_adj(z):
    bi, bj = 2560, 2048

    def body(a_ref, b_ref, o_ref):
        prod = lax.dot_general(
            a_ref[...], b_ref[...], (((1,), (1,)), ((), ())),
            preferred_element_type=jnp.float32)
        o_ref[...] = 0.5 * jnp.tanh(0.5 * prod) + 0.5

    return pl.pallas_call(
        body,
        grid=(pl.cdiv(_N, bi), pl.cdiv(_N, bj)),
        in_specs=[
            pl.BlockSpec((bi, 32), lambda i, j: (i, 0)),
            pl.BlockSpec((bj, 32), lambda i, j: (j, 0)),
        ],
        out_specs=pl.BlockSpec((bi, bj), lambda i, j: (i, j)),
        out_shape=jax.ShapeDtypeStruct((_N, _N), jnp.float32),
    )(z, z)


def kernel(x_features, edge_index, W1, b1, Wmu, bmu, Wlv, blv, W2, b2, W3, b3):
    ei = edge_index.astype(jnp.int32)
    pad = jnp.full((_EPAD - _E,), _NPAD - 1, jnp.int32)
    s3 = jnp.concatenate([ei[0], pad]).reshape(_NW, _NCH, _CH)
    d3 = jnp.concatenate([ei[1], pad]).reshape(_NW, _NCH, _CH)
    e4 = jnp.stack([s3, d3], axis=2)
    x_pad = jnp.pad(x_features, ((0, _NPAD - _N), (0, 0)))
    wcat = jnp.concatenate([Wmu, Wlv], axis=1)
    bcat = jnp.concatenate([bmu, blv]).reshape(1, 64)

    deg2 = _deg(e4)
    xw1 = _tc_xw1(x_pad, W1)
    y1, dinv = _tc_scale1(deg2, xw1)
    acc1 = _agg64(e4, y1)
    u2 = _tc_elt(acc1, y1, dinv, b1.reshape(1, 64))
    acc2 = _agg64(e4, u2)
    mu, logvar, u3 = _tc_latent(acc2, u2, dinv, bcat, wcat)
    acc3 = _agg32(e4, u3)
    u4 = _tc_dec1(acc3, u3, dinv, W2, b2.reshape(1, 64))
    acc4 = _agg64(e4, u4)
    x_pred = _tc_dec2(acc4, u4, dinv, W3, b3.reshape(1, 128))
    A_pred = _tc_adj(mu)
    z = mu
    return (A_pred, mu, logvar, z, x_pred)
